# Initial kernel scaffold; baseline (speedup 1.0000x reference)
import jax
import jax.numpy as jnp
from jax.experimental import pallas as pl


def kernel(cube, step_count, embed_table, W, b):
    raise NotImplementedError("write your pallas kernel here")


# speedup vs baseline: 3.4336x; 3.4336x over previous
import functools

import jax
import jax.numpy as jnp
from jax import lax
from jax.experimental import pallas as pl
from jax.experimental.pallas import tpu as pltpu
from jax.experimental.pallas import tpu_sc as plsc

B_TOTAL = 16384
N_STICKERS = 54
EMBED_D = 64
OBS_W = N_STICKERS * EMBED_D
STEP_W = 32
OUT_W = OBS_W + STEP_W
TIME_LIMIT = 200.0

NC = 2
NS = 16
NW = NC * NS
BPW = B_TOTAL // NW
NG = BPW // 16


def _lane_bcast(vec, lane):
    return vec[jnp.full((16,), lane, jnp.int32)]


def _sc_body(cube_hbm, step_hbm, table_hbm, w_hbm, b_hbm, out_hbm,
             cube_v, step_v, table_v, w_v, b_v, buf, sbuf,
             sem0, sem1, sem_s):
    sems = (sem0, sem1)
    wid = lax.axis_index("s") * NC + lax.axis_index("c")
    bcol = wid * BPW

    pltpu.sync_copy(cube_hbm.at[:, pl.ds(bcol, BPW)], cube_v)
    pltpu.sync_copy(step_hbm.at[pl.ds(bcol, BPW)], step_v)
    pltpu.sync_copy(table_hbm, table_v)
    pltpu.sync_copy(w_hbm, w_v)
    pltpu.sync_copy(b_hbm, b_v)

    def build(bb, s):
        def per_group(g, _):
            base = cube_v[s, pl.ds(g * 16, 16)] * EMBED_D
            for d in range(EMBED_D):
                val = plsc.load_gather(table_v, [base + d])
                buf[bb, d, pl.ds(g * 16, 16)] = val
            return 0

        lax.fori_loop(0, NG, per_group, 0)

    def start_out(bb, s):
        pltpu.async_copy(
            buf.at[bb],
            out_hbm.at[pl.ds(s * EMBED_D, EMBED_D), pl.ds(bcol, BPW)],
            sems[bb])

    def wait_out(bb):
        pltpu.make_async_copy(
            buf.at[bb],
            out_hbm.at[pl.ds(0, EMBED_D), pl.ds(bcol, BPW)],
            sems[bb]).wait()

    for bb in range(2):
        build(bb, bb)
        start_out(bb, bb)

    def pair(i, _):
        s0 = 2 + i * 2
        for bb in range(2):
            wait_out(bb)
            build(bb, s0 + bb)
            start_out(bb, s0 + bb)
        return 0

    lax.fori_loop(0, (N_STICKERS - 2) // 2, pair, 0)

    wv0 = w_v[pl.ds(0, 16)]
    wv1 = w_v[pl.ds(16, 16)]
    bv0 = b_v[pl.ds(0, 16)]
    bv1 = b_v[pl.ds(16, 16)]
    ws = [_lane_bcast(wv0, k) for k in range(16)] + \
         [_lane_bcast(wv1, k) for k in range(16)]
    bs = [_lane_bcast(bv0, k) for k in range(16)] + \
         [_lane_bcast(bv1, k) for k in range(16)]

    def step_group(g, _):
        sv = step_v[pl.ds(g * 16, 16)].astype(jnp.float32) / TIME_LIMIT
        for k in range(STEP_W):
            sbuf[k, pl.ds(g * 16, 16)] = sv * ws[k] + bs[k]
        return 0

    lax.fori_loop(0, NG, step_group, 0)
    pltpu.async_copy(
        sbuf,
        out_hbm.at[pl.ds(OBS_W, STEP_W), pl.ds(bcol, BPW)],
        sem_s)

    for bb in range(2):
        wait_out(bb)
    pltpu.make_async_copy(
        sbuf,
        out_hbm.at[pl.ds(OBS_W, STEP_W), pl.ds(bcol, BPW)],
        sem_s).wait()


def kernel(cube, step_count, embed_table, W, b):
    assert cube.shape == (B_TOTAL, 6, 3, 3)
    cube_t = cube.reshape(B_TOTAL, N_STICKERS).T
    table_flat = embed_table.reshape(-1)
    w_row = W.reshape(-1)

    mesh = plsc.VectorSubcoreMesh(core_axis_name="c", subcore_axis_name="s")
    run = functools.partial(
        pl.kernel,
        out_type=jax.ShapeDtypeStruct((OUT_W, B_TOTAL), jnp.float32),
        mesh=mesh,
        compiler_params=pltpu.CompilerParams(needs_layout_passes=False),
        scratch_types=[
            pltpu.VMEM((N_STICKERS, BPW), jnp.int32),
            pltpu.VMEM((BPW,), jnp.int32),
            pltpu.VMEM((6 * EMBED_D,), jnp.float32),
            pltpu.VMEM((STEP_W,), jnp.float32),
            pltpu.VMEM((STEP_W,), jnp.float32),
            pltpu.VMEM((2, EMBED_D, BPW), jnp.float32),
            pltpu.VMEM((STEP_W, BPW), jnp.float32),
            pltpu.SemaphoreType.DMA,
            pltpu.SemaphoreType.DMA,
            pltpu.SemaphoreType.DMA,
        ],
    )(_sc_body)
    out_t = run(cube_t, step_count, table_flat, w_row, b)
    return out_t.T
